# baseline (speedup 1.0000x reference)
import functools

import jax
import jax.numpy as jnp
from jax import lax
from jax.experimental import pallas as pl
from jax.experimental.pallas import tpu as pltpu
from jax.experimental.pallas import tpu_sc as plsc

N = 10000
E = 320000
D = 128
CH = 128

DEG_NCH = 80
DEG_ROWS = 10112
DEG_SLICE = 632

ACH = 64
AGG_NCH = 320
SCH = 128
AGG_NSCH = 160
AGG_NBUF = 4
ACC_ROWS = 10112
ACC_SLICE = 632

_mesh = plsc.VectorSubcoreMesh(core_axis_name="c", subcore_axis_name="s")


def _deg_body(col_hbm, ones_hbm, zeros_hbm, out_hbm, tab, onesv, cbuf, sd):
    c = lax.axis_index("c")
    s = lax.axis_index("s")
    wid = c * 16 + s
    pltpu.sync_copy(zeros_hbm, onesv)
    for off, sz in ((0, 128), (128, 128), (256, 128), (384, 128), (512, 120)):
        pltpu.sync_copy(onesv.at[pl.ds(0, sz)],
                        tab.at[pl.ds(s * DEG_SLICE + off, sz)])
    pltpu.sync_copy(ones_hbm, onesv)
    plsc.subcore_barrier()
    pltpu.sync_copy(col_hbm.at[pl.ds(wid * DEG_NCH, DEG_NCH)], cbuf)
    for j in range(DEG_NCH):
        if j >= 4:
            pltpu.make_async_copy(onesv, tab.at[cbuf.at[j - 4]], sd).wait()
        pltpu.async_copy(onesv, tab.at[cbuf.at[j]], sd, add=True)
    for j in range(DEG_NCH - 4, DEG_NCH):
        pltpu.make_async_copy(onesv, tab.at[cbuf.at[j]], sd).wait()
    plsc.subcore_barrier()
    pltpu.sync_copy(tab.at[pl.ds(s * DEG_SLICE, DEG_SLICE)],
                    out_hbm.at[pl.ds(c * DEG_ROWS + s * DEG_SLICE, DEG_SLICE)])


_deg_call = pl.kernel(
    _deg_body,
    out_type=jax.ShapeDtypeStruct((2 * DEG_ROWS, D), jnp.float32),
    mesh=_mesh,
    scratch_types=[
        pltpu.VMEM_SHARED((DEG_ROWS, D), jnp.float32),
        pltpu.VMEM((CH, D), jnp.float32),
        pltpu.VMEM((DEG_NCH, CH), jnp.int32),
        pltpu.SemaphoreType.DMA,
    ],
)


AGG_NST = 5
AGG_CPS = AGG_NCH // AGG_NST


def _agg_body(g_hbm, row_hbm, col_hbm, zeros_hbm, out_hbm,
              acc, rbuf, cbuf, d01, d23, sg0, sg1, sg2, sg3, ss0, ss1):
    c = lax.axis_index("c")
    s = lax.axis_index("s")
    halves = (d01.at[pl.ds(0, ACH)], d01.at[pl.ds(ACH, ACH)],
              d23.at[pl.ds(0, ACH)], d23.at[pl.ds(ACH, ACH)])
    sgs = (sg0, sg1, sg2, sg3)
    d0 = halves[0]
    n_full = (ACC_SLICE // ACH) * ACH
    rem = ACC_SLICE - n_full
    pltpu.sync_copy(zeros_hbm, d0)
    for off in range(0, n_full, ACH):
        pltpu.sync_copy(d0, acc.at[pl.ds(s * ACC_SLICE + off, ACH)])
    if rem:
        pltpu.sync_copy(d0.at[pl.ds(0, rem)],
                        acc.at[pl.ds(s * ACC_SLICE + n_full, rem)])
    plsc.subcore_barrier()

    SCPS = AGG_CPS // 2
    for st in range(AGG_NST):
        pltpu.sync_copy(
            row_hbm.at[pl.ds((c * 16 + s) * AGG_NCH + st * AGG_CPS, AGG_CPS)],
            rbuf)
        pltpu.sync_copy(
            col_hbm.at[pl.ds(s * AGG_NSCH + st * SCPS, SCPS)], cbuf)
        for b in range(4):
            pltpu.async_copy(g_hbm.at[rbuf.at[b]], halves[b], sgs[b])

        def body(i, carry):
            j0 = 4 * i
            p0 = 2 * i
            for h in (0, 1):
                pltpu.make_async_copy(g_hbm.at[rbuf.at[j0 + 2 * h]],
                                      halves[2 * h], sgs[2 * h]).wait()
                pltpu.make_async_copy(g_hbm.at[rbuf.at[j0 + 2 * h + 1]],
                                      halves[2 * h + 1], sgs[2 * h + 1]).wait()
                pltpu.async_copy((d01, d23)[h], acc.at[cbuf.at[p0 + h]],
                                 (ss0, ss1)[h], add=True)
            for h in (0, 1):
                pltpu.make_async_copy((d01, d23)[h], acc.at[cbuf.at[p0 + h]],
                                      (ss0, ss1)[h]).wait()

                @pl.when(i < AGG_CPS // 4 - 1)
                def _(j0=j0, h=h):
                    pltpu.async_copy(g_hbm.at[rbuf.at[j0 + 4 + 2 * h]],
                                     halves[2 * h], sgs[2 * h])
                    pltpu.async_copy(g_hbm.at[rbuf.at[j0 + 4 + 2 * h + 1]],
                                     halves[2 * h + 1], sgs[2 * h + 1])

            return carry

        lax.fori_loop(0, AGG_CPS // 4, body, 0)

    plsc.subcore_barrier()
    pltpu.sync_copy(acc.at[pl.ds(s * ACC_SLICE, ACC_SLICE)],
                    out_hbm.at[pl.ds(c * ACC_ROWS + s * ACC_SLICE, ACC_SLICE)])


_agg_call = pl.kernel(
    _agg_body,
    out_type=jax.ShapeDtypeStruct((2 * ACC_ROWS, D), jnp.float32),
    mesh=_mesh,
    scratch_types=[
        pltpu.VMEM_SHARED((ACC_ROWS, D), jnp.float32),
        pltpu.VMEM((AGG_CPS, ACH), jnp.int32),
        pltpu.VMEM((AGG_CPS // 2, SCH), jnp.int32),
        pltpu.VMEM((SCH, D), jnp.float32),
        pltpu.VMEM((SCH, D), jnp.float32),
    ] + [pltpu.SemaphoreType.DMA] * 6,
)


def _dis_from_tabs(t0_ref, t1_ref):
    deg = t0_ref[0][:, 0:1] + t1_ref[0][:, 0:1] + 1.0
    return lax.rsqrt(deg)


def _tc1_body(x_ref, w_ref, b_ref, t0_ref, t1_ref, o_ref, dis_ref):
    dis = _dis_from_tabs(t0_ref, t1_ref)
    h = jnp.dot(x_ref[0], w_ref[0].T, preferred_element_type=jnp.float32)
    o_ref[0] = dis * (h + b_ref[0, 0][None, :])
    dis_ref[...] = dis


def _tc2_body(a_ref, g_ref, dis_ref, wo_ref, bo_ref, o_ref):
    dis = dis_ref[...]
    y = jnp.maximum(dis * (a_ref[0] + g_ref[0]), 0.0)
    h2 = jnp.dot(y, wo_ref[...].T, preferred_element_type=jnp.float32)
    o_ref[0] = dis * (h2 + bo_ref[0][None, :])


def _tc3_body(a_ref, g_ref, dis_ref, w_ref, o_ref):
    dis = dis_ref[...]
    o = dis * (a_ref[0] + g_ref[0])
    e0 = jnp.exp(w_ref[0])
    e1 = jnp.exp(w_ref[1])
    scale = jnp.where(pl.program_id(0) == 0, e0, e1) / (e0 + e1)
    o_ref[...] = scale * o


_NB = 10
_BR = N // _NB

_spec_x = pl.BlockSpec((1, _BR, D), lambda c, j: (c, j, 0))
_spec_tab0 = pl.BlockSpec((1, _BR, D), lambda c, j: (0, j, 0))
_spec_tab1 = pl.BlockSpec((1, _BR, D), lambda c, j: (1, j, 0))

_tc1 = pl.pallas_call(
    _tc1_body,
    grid=(2, _NB),
    in_specs=[
        _spec_x,
        pl.BlockSpec((1, D, D), lambda c, j: (c, 0, 0)),
        pl.BlockSpec((1, 1, D), lambda c, j: (c, 0, 0)),
        _spec_tab0,
        _spec_tab1,
    ],
    out_specs=[_spec_x, pl.BlockSpec((_BR, 1), lambda c, j: (j, 0))],
    out_shape=[jax.ShapeDtypeStruct((2, N, D), jnp.float32),
               jax.ShapeDtypeStruct((N, 1), jnp.float32)],
)

_spec_dis = pl.BlockSpec((_BR, 1), lambda c, j: (j, 0))

_tc2 = pl.pallas_call(
    _tc2_body,
    grid=(2, _NB),
    in_specs=[
        _spec_x,
        _spec_x,
        _spec_dis,
        pl.BlockSpec((D, D), lambda c, j: (0, 0)),
        pl.BlockSpec((1, D), lambda c, j: (0, 0)),
    ],
    out_specs=_spec_x,
    out_shape=jax.ShapeDtypeStruct((2, N, D), jnp.float32),
)

_tc3 = pl.pallas_call(
    _tc3_body,
    grid=(2, _NB),
    in_specs=[
        _spec_x,
        _spec_x,
        _spec_dis,
        pl.BlockSpec(memory_space=pltpu.SMEM),
    ],
    out_specs=pl.BlockSpec((_BR, D), lambda c, j: (j, c)),
    out_shape=jax.ShapeDtypeStruct((N, 2 * D), jnp.float32),
)


def kernel(x, edge_index, W1, b1, W2, b2, Wo, bo, weight):
    row = edge_index[0]
    col = edge_index[1]

    col_d = jnp.pad(col.reshape(32, E // 32), ((0, 0), (0, DEG_NCH * CH - E // 32)),
                    constant_values=N).reshape(32 * DEG_NCH, CH)
    r16 = jnp.pad(row.reshape(16, E // 16), ((0, 0), (0, AGG_NCH * ACH - E // 16)))
    c16 = jnp.pad(col.reshape(16, E // 16), ((0, 0), (0, AGG_NCH * ACH - E // 16)),
                  constant_values=N)
    row2 = jnp.concatenate([r16, r16 + N], axis=0).reshape(32 * AGG_NCH, ACH)
    colh = c16.reshape(16 * AGG_NSCH, SCH)

    ones128 = jnp.ones((CH, D), jnp.float32)
    zeros128 = jnp.zeros((CH, D), jnp.float32)
    zeros64 = jnp.zeros((ACH, D), jnp.float32)

    degtab = _deg_call(col_d, ones128, zeros128).reshape(2, DEG_ROWS, D)

    Wst = jnp.stack([W1, W2])
    bst = jnp.stack([b1, b2]).reshape(2, 1, D)
    G1, dis = _tc1(x, Wst, bst, degtab, degtab)
    A1 = _agg_call(G1.reshape(2 * N, D), row2, colh, zeros64)
    Agg1 = A1.reshape(2, ACC_ROWS, D)

    G2 = _tc2(Agg1, G1, dis, Wo, bo.reshape(1, D))
    A2 = _agg_call(G2.reshape(2 * N, D), row2, colh, zeros64)
    Agg2 = A2.reshape(2, ACC_ROWS, D)

    return _tc3(Agg2, G2, dis, weight)


# speedup vs baseline: 1.0875x; 1.0875x over previous
import functools

import jax
import jax.numpy as jnp
from jax import lax
from jax.experimental import pallas as pl
from jax.experimental.pallas import tpu as pltpu
from jax.experimental.pallas import tpu_sc as plsc

N = 10000
E = 320000
D = 128
CH = 128

DEG_NCH = 80
DEG_ROWS = 10112
DEG_SLICE = 632

ACH = 64
AGG_NCH = 320
AGG_NBUF = 4
ACC_ROWS = 10112
ACC_SLICE = 632

_mesh = plsc.VectorSubcoreMesh(core_axis_name="c", subcore_axis_name="s")


def _deg_body(col_hbm, ones_hbm, zeros_hbm, out_hbm, tab, onesv, cbuf, sd):
    c = lax.axis_index("c")
    s = lax.axis_index("s")
    wid = c * 16 + s
    pltpu.sync_copy(zeros_hbm, onesv)
    for off, sz in ((0, 128), (128, 128), (256, 128), (384, 128), (512, 120)):
        pltpu.sync_copy(onesv.at[pl.ds(0, sz)],
                        tab.at[pl.ds(s * DEG_SLICE + off, sz)])
    pltpu.sync_copy(ones_hbm, onesv)
    plsc.subcore_barrier()
    pltpu.sync_copy(col_hbm.at[pl.ds(wid * DEG_NCH, DEG_NCH)], cbuf)
    for j in range(DEG_NCH):
        if j >= 4:
            pltpu.make_async_copy(onesv, tab.at[cbuf.at[j - 4]], sd).wait()
        pltpu.async_copy(onesv, tab.at[cbuf.at[j]], sd, add=True)
    for j in range(DEG_NCH - 4, DEG_NCH):
        pltpu.make_async_copy(onesv, tab.at[cbuf.at[j]], sd).wait()
    plsc.subcore_barrier()
    pltpu.sync_copy(tab.at[pl.ds(s * DEG_SLICE, DEG_SLICE)],
                    out_hbm.at[pl.ds(c * DEG_ROWS + s * DEG_SLICE, DEG_SLICE)])


_deg_call = pl.kernel(
    _deg_body,
    out_type=jax.ShapeDtypeStruct((2 * DEG_ROWS, D), jnp.float32),
    mesh=_mesh,
    scratch_types=[
        pltpu.VMEM_SHARED((DEG_ROWS, D), jnp.float32),
        pltpu.VMEM((CH, D), jnp.float32),
        pltpu.VMEM((DEG_NCH, CH), jnp.int32),
        pltpu.SemaphoreType.DMA,
    ],
)


AGG_NST = 5
AGG_CPS = AGG_NCH // AGG_NST


def _agg_body(g_hbm, row_hbm, col_hbm, zeros_hbm, out_hbm,
              acc, rbuf, cbuf, *rest):
    c = lax.axis_index("c")
    s = lax.axis_index("s")
    bufs = rest[:AGG_NBUF]
    sgs = rest[AGG_NBUF:2 * AGG_NBUF]
    sss = rest[2 * AGG_NBUF:3 * AGG_NBUF]
    d0 = bufs[0]
    n_full = (ACC_SLICE // ACH) * ACH
    rem = ACC_SLICE - n_full
    pltpu.sync_copy(zeros_hbm, d0)
    for off in range(0, n_full, ACH):
        pltpu.sync_copy(d0, acc.at[pl.ds(s * ACC_SLICE + off, ACH)])
    if rem:
        pltpu.sync_copy(d0.at[pl.ds(0, rem)],
                        acc.at[pl.ds(s * ACC_SLICE + n_full, rem)])
    plsc.subcore_barrier()

    for st in range(AGG_NST):
        pltpu.sync_copy(
            row_hbm.at[pl.ds((c * 16 + s) * AGG_NCH + st * AGG_CPS, AGG_CPS)],
            rbuf)
        pltpu.sync_copy(
            col_hbm.at[pl.ds(s * AGG_NCH + st * AGG_CPS, AGG_CPS)], cbuf)
        for b in range(AGG_NBUF):
            pltpu.async_copy(g_hbm.at[rbuf.at[b]], bufs[b], sgs[b])

        def body(i, carry):
            for b in range(AGG_NBUF):
                j = AGG_NBUF * i + b
                pltpu.make_async_copy(g_hbm.at[rbuf.at[j]],
                                      bufs[b], sgs[b]).wait()
                pltpu.async_copy(bufs[b], acc.at[cbuf.at[j]], sss[b],
                                 add=True)
            for b in range(AGG_NBUF):
                j = AGG_NBUF * i + b
                pltpu.make_async_copy(bufs[b], acc.at[cbuf.at[j]],
                                      sss[b]).wait()

                @pl.when(i < AGG_CPS // AGG_NBUF - 1)
                def _(j=j, b=b):
                    pltpu.async_copy(g_hbm.at[rbuf.at[j + AGG_NBUF]],
                                     bufs[b], sgs[b])

            return carry

        lax.fori_loop(0, AGG_CPS // AGG_NBUF, body, 0)

    plsc.subcore_barrier()
    pltpu.sync_copy(acc.at[pl.ds(s * ACC_SLICE, ACC_SLICE)],
                    out_hbm.at[pl.ds(c * ACC_ROWS + s * ACC_SLICE, ACC_SLICE)])


_agg_call = pl.kernel(
    _agg_body,
    out_type=jax.ShapeDtypeStruct((2 * ACC_ROWS, D), jnp.float32),
    mesh=_mesh,
    scratch_types=[
        pltpu.VMEM_SHARED((ACC_ROWS, D), jnp.float32),
        pltpu.VMEM((AGG_CPS, ACH), jnp.int32),
        pltpu.VMEM((AGG_CPS, ACH), jnp.int32),
    ] + [pltpu.VMEM((ACH, D), jnp.float32)] * AGG_NBUF
      + [pltpu.SemaphoreType.DMA] * (2 * AGG_NBUF),
)


def _dis_from_tabs(t0_ref, t1_ref):
    deg = t0_ref[0][:, 0:1] + t1_ref[0][:, 0:1] + 1.0
    return lax.rsqrt(deg)


def _tc1_body(x_ref, w_ref, b_ref, t0_ref, t1_ref, o_ref, dis_ref):
    dis = _dis_from_tabs(t0_ref, t1_ref)
    h = jnp.dot(x_ref[0], w_ref[0].T, preferred_element_type=jnp.float32)
    o_ref[0] = dis * (h + b_ref[0, 0][None, :])
    dis_ref[...] = dis


def _tc2_body(a_ref, g_ref, dis_ref, wo_ref, bo_ref, o_ref):
    dis = dis_ref[...]
    y = jnp.maximum(dis * (a_ref[0] + g_ref[0]), 0.0)
    h2 = jnp.dot(y, wo_ref[...].T, preferred_element_type=jnp.float32)
    o_ref[0] = dis * (h2 + bo_ref[0][None, :])


def _tc3_body(a_ref, g_ref, dis_ref, w_ref, o_ref):
    dis = dis_ref[...]
    o = dis * (a_ref[0] + g_ref[0])
    e0 = jnp.exp(w_ref[0])
    e1 = jnp.exp(w_ref[1])
    scale = jnp.where(pl.program_id(0) == 0, e0, e1) / (e0 + e1)
    o_ref[...] = scale * o


_NB = 10
_BR = N // _NB

_spec_x = pl.BlockSpec((1, _BR, D), lambda c, j: (c, j, 0))
_spec_tab0 = pl.BlockSpec((1, _BR, D), lambda c, j: (0, j, 0))
_spec_tab1 = pl.BlockSpec((1, _BR, D), lambda c, j: (1, j, 0))

_tc1 = pl.pallas_call(
    _tc1_body,
    grid=(2, _NB),
    in_specs=[
        _spec_x,
        pl.BlockSpec((1, D, D), lambda c, j: (c, 0, 0)),
        pl.BlockSpec((1, 1, D), lambda c, j: (c, 0, 0)),
        _spec_tab0,
        _spec_tab1,
    ],
    out_specs=[_spec_x, pl.BlockSpec((_BR, 1), lambda c, j: (j, 0))],
    out_shape=[jax.ShapeDtypeStruct((2, N, D), jnp.float32),
               jax.ShapeDtypeStruct((N, 1), jnp.float32)],
)

_spec_dis = pl.BlockSpec((_BR, 1), lambda c, j: (j, 0))

_tc2 = pl.pallas_call(
    _tc2_body,
    grid=(2, _NB),
    in_specs=[
        _spec_x,
        _spec_x,
        _spec_dis,
        pl.BlockSpec((D, D), lambda c, j: (0, 0)),
        pl.BlockSpec((1, D), lambda c, j: (0, 0)),
    ],
    out_specs=_spec_x,
    out_shape=jax.ShapeDtypeStruct((2, N, D), jnp.float32),
)

_tc3 = pl.pallas_call(
    _tc3_body,
    grid=(2, _NB),
    in_specs=[
        _spec_x,
        _spec_x,
        _spec_dis,
        pl.BlockSpec(memory_space=pltpu.SMEM),
    ],
    out_specs=pl.BlockSpec((_BR, D), lambda c, j: (j, c)),
    out_shape=jax.ShapeDtypeStruct((N, 2 * D), jnp.float32),
)


def kernel(x, edge_index, W1, b1, W2, b2, Wo, bo, weight):
    row = edge_index[0]
    col = edge_index[1]

    col_d = jnp.pad(col.reshape(32, E // 32), ((0, 0), (0, DEG_NCH * CH - E // 32)),
                    constant_values=N).reshape(32 * DEG_NCH, CH)
    r16 = jnp.pad(row.reshape(16, E // 16), ((0, 0), (0, AGG_NCH * ACH - E // 16)))
    c16 = jnp.pad(col.reshape(16, E // 16), ((0, 0), (0, AGG_NCH * ACH - E // 16)),
                  constant_values=N)
    row2 = jnp.concatenate([r16, r16 + N], axis=0).reshape(32 * AGG_NCH, ACH)
    colh = c16.reshape(16 * AGG_NCH, ACH)

    ones128 = jnp.ones((CH, D), jnp.float32)
    zeros128 = jnp.zeros((CH, D), jnp.float32)
    zeros64 = jnp.zeros((ACH, D), jnp.float32)

    degtab = _deg_call(col_d, ones128, zeros128).reshape(2, DEG_ROWS, D)

    Wst = jnp.stack([W1, W2])
    bst = jnp.stack([b1, b2]).reshape(2, 1, D)
    G1, dis = _tc1(x, Wst, bst, degtab, degtab)
    A1 = _agg_call(G1.reshape(2 * N, D), row2, colh, zeros64)
    Agg1 = A1.reshape(2, ACC_ROWS, D)

    G2 = _tc2(Agg1, G1, dis, Wo, bo.reshape(1, D))
    A2 = _agg_call(G2.reshape(2 * N, D), row2, colh, zeros64)
    Agg2 = A2.reshape(2, ACC_ROWS, D)

    return _tc3(Agg2, G2, dis, weight)


# speedup vs baseline: 1.0891x; 1.0015x over previous
import functools

import jax
import jax.numpy as jnp
from jax import lax
from jax.experimental import pallas as pl
from jax.experimental.pallas import tpu as pltpu
from jax.experimental.pallas import tpu_sc as plsc

N = 10000
E = 320000
D = 128
CH = 128

DEG_NCH = 80
DEG_ROWS = 10112
DEG_SLICE = 632

ACH = 64
AGG_NCH = 320
AGG_NBUF = 4
ACC_ROWS = 10112
ACC_SLICE = 632

_mesh = plsc.VectorSubcoreMesh(core_axis_name="c", subcore_axis_name="s")


def _deg_body(col_hbm, ones_hbm, zeros_hbm, out_hbm, tab, onesv, cbuf, sd):
    c = lax.axis_index("c")
    s = lax.axis_index("s")
    wid = c * 16 + s
    pltpu.sync_copy(zeros_hbm, onesv)
    for off, sz in ((0, 128), (128, 128), (256, 128), (384, 128), (512, 120)):
        pltpu.sync_copy(onesv.at[pl.ds(0, sz)],
                        tab.at[pl.ds(s * DEG_SLICE + off, sz)])
    pltpu.sync_copy(ones_hbm, onesv)
    plsc.subcore_barrier()
    pltpu.sync_copy(col_hbm.at[pl.ds(wid * DEG_NCH, DEG_NCH)], cbuf)
    for j in range(DEG_NCH):
        if j >= 4:
            pltpu.make_async_copy(onesv, tab.at[cbuf.at[j - 4]], sd).wait()
        pltpu.async_copy(onesv, tab.at[cbuf.at[j]], sd, add=True)
    for j in range(DEG_NCH - 4, DEG_NCH):
        pltpu.make_async_copy(onesv, tab.at[cbuf.at[j]], sd).wait()
    plsc.subcore_barrier()
    pltpu.sync_copy(tab.at[pl.ds(s * DEG_SLICE, DEG_SLICE)],
                    out_hbm.at[pl.ds(c * DEG_ROWS + s * DEG_SLICE, DEG_SLICE)])


_deg_call = pl.kernel(
    _deg_body,
    out_type=jax.ShapeDtypeStruct((2 * DEG_ROWS, D), jnp.float32),
    mesh=_mesh,
    scratch_types=[
        pltpu.VMEM_SHARED((DEG_ROWS, D), jnp.float32),
        pltpu.VMEM((CH, D), jnp.float32),
        pltpu.VMEM((DEG_NCH, CH), jnp.int32),
        pltpu.SemaphoreType.DMA,
    ],
)


AGG_NST = 10
AGG_CPS = AGG_NCH // AGG_NST


def _agg_body(g_hbm, row_hbm, col_hbm, zeros_hbm, out_hbm,
              acc, rbuf0, cbuf0, rbuf1, cbuf1, si, *rest):
    c = lax.axis_index("c")
    s = lax.axis_index("s")
    bufs = rest[:AGG_NBUF]
    sgs = rest[AGG_NBUF:2 * AGG_NBUF]
    sss = rest[2 * AGG_NBUF:3 * AGG_NBUF]
    d0 = bufs[0]
    n_full = (ACC_SLICE // ACH) * ACH
    rem = ACC_SLICE - n_full
    pltpu.sync_copy(zeros_hbm, d0)
    for off in range(0, n_full, ACH):
        pltpu.sync_copy(d0, acc.at[pl.ds(s * ACC_SLICE + off, ACH)])
    if rem:
        pltpu.sync_copy(d0.at[pl.ds(0, rem)],
                        acc.at[pl.ds(s * ACC_SLICE + n_full, rem)])
    plsc.subcore_barrier()

    rbase = (c * 16 + s) * AGG_NCH
    cbase = s * AGG_NCH
    pltpu.sync_copy(row_hbm.at[pl.ds(rbase, AGG_CPS)], rbuf0)
    pltpu.sync_copy(col_hbm.at[pl.ds(cbase, AGG_CPS)], cbuf0)
    for st in range(AGG_NST):
        rbuf, cbuf = (rbuf0, cbuf0) if st % 2 == 0 else (rbuf1, cbuf1)
        nr, ncb = (rbuf1, cbuf1) if st % 2 == 0 else (rbuf0, cbuf0)
        if st > 0:
            pltpu.make_async_copy(
                row_hbm.at[pl.ds(rbase + st * AGG_CPS, AGG_CPS)],
                rbuf, si).wait()
            pltpu.make_async_copy(
                col_hbm.at[pl.ds(cbase + st * AGG_CPS, AGG_CPS)],
                cbuf, si).wait()
        if st < AGG_NST - 1:
            pltpu.async_copy(
                row_hbm.at[pl.ds(rbase + (st + 1) * AGG_CPS, AGG_CPS)],
                nr, si)
            pltpu.async_copy(
                col_hbm.at[pl.ds(cbase + (st + 1) * AGG_CPS, AGG_CPS)],
                ncb, si)
        for b in range(AGG_NBUF):
            pltpu.async_copy(g_hbm.at[rbuf.at[b]], bufs[b], sgs[b])

        def body(i, carry, rbuf=rbuf, cbuf=cbuf):
            for b in range(AGG_NBUF):
                j = AGG_NBUF * i + b
                pltpu.make_async_copy(g_hbm.at[rbuf.at[j]],
                                      bufs[b], sgs[b]).wait()
                pltpu.async_copy(bufs[b], acc.at[cbuf.at[j]], sss[b],
                                 add=True)
            for b in range(AGG_NBUF):
                j = AGG_NBUF * i + b
                pltpu.make_async_copy(bufs[b], acc.at[cbuf.at[j]],
                                      sss[b]).wait()

                @pl.when(i < AGG_CPS // AGG_NBUF - 1)
                def _(j=j, b=b):
                    pltpu.async_copy(g_hbm.at[rbuf.at[j + AGG_NBUF]],
                                     bufs[b], sgs[b])

            return carry

        lax.fori_loop(0, AGG_CPS // AGG_NBUF, body, 0)

    plsc.subcore_barrier()
    pltpu.sync_copy(acc.at[pl.ds(s * ACC_SLICE, ACC_SLICE)],
                    out_hbm.at[pl.ds(c * ACC_ROWS + s * ACC_SLICE, ACC_SLICE)])


_agg_call = pl.kernel(
    _agg_body,
    out_type=jax.ShapeDtypeStruct((2 * ACC_ROWS, D), jnp.float32),
    mesh=_mesh,
    scratch_types=[
        pltpu.VMEM_SHARED((ACC_ROWS, D), jnp.float32),
        pltpu.VMEM((AGG_CPS, ACH), jnp.int32),
        pltpu.VMEM((AGG_CPS, ACH), jnp.int32),
        pltpu.VMEM((AGG_CPS, ACH), jnp.int32),
        pltpu.VMEM((AGG_CPS, ACH), jnp.int32),
        pltpu.SemaphoreType.DMA,
    ] + [pltpu.VMEM((ACH, D), jnp.float32)] * AGG_NBUF
      + [pltpu.SemaphoreType.DMA] * (2 * AGG_NBUF),
)


def _dis_from_tabs(t0_ref, t1_ref):
    deg = t0_ref[0][:, 0:1] + t1_ref[0][:, 0:1] + 1.0
    return lax.rsqrt(deg)


def _tc1_body(x_ref, w_ref, b_ref, t0_ref, t1_ref, o_ref, dis_ref):
    dis = _dis_from_tabs(t0_ref, t1_ref)
    h = jnp.dot(x_ref[0], w_ref[0].T, preferred_element_type=jnp.float32)
    o_ref[0] = dis * (h + b_ref[0, 0][None, :])
    dis_ref[...] = dis


def _tc2_body(a_ref, g_ref, dis_ref, wo_ref, bo_ref, o_ref):
    dis = dis_ref[...]
    y = jnp.maximum(dis * (a_ref[0] + g_ref[0]), 0.0)
    h2 = jnp.dot(y, wo_ref[...].T, preferred_element_type=jnp.float32)
    o_ref[0] = dis * (h2 + bo_ref[0][None, :])


def _tc3_body(a_ref, g_ref, dis_ref, w_ref, o_ref):
    dis = dis_ref[...]
    o = dis * (a_ref[0] + g_ref[0])
    e0 = jnp.exp(w_ref[0])
    e1 = jnp.exp(w_ref[1])
    scale = jnp.where(pl.program_id(0) == 0, e0, e1) / (e0 + e1)
    o_ref[...] = scale * o


_NB = 10
_BR = N // _NB

_spec_x = pl.BlockSpec((1, _BR, D), lambda c, j: (c, j, 0))
_spec_tab0 = pl.BlockSpec((1, _BR, D), lambda c, j: (0, j, 0))
_spec_tab1 = pl.BlockSpec((1, _BR, D), lambda c, j: (1, j, 0))

_tc1 = pl.pallas_call(
    _tc1_body,
    grid=(2, _NB),
    in_specs=[
        _spec_x,
        pl.BlockSpec((1, D, D), lambda c, j: (c, 0, 0)),
        pl.BlockSpec((1, 1, D), lambda c, j: (c, 0, 0)),
        _spec_tab0,
        _spec_tab1,
    ],
    out_specs=[_spec_x, pl.BlockSpec((_BR, 1), lambda c, j: (j, 0))],
    out_shape=[jax.ShapeDtypeStruct((2, N, D), jnp.float32),
               jax.ShapeDtypeStruct((N, 1), jnp.float32)],
)

_spec_dis = pl.BlockSpec((_BR, 1), lambda c, j: (j, 0))

_tc2 = pl.pallas_call(
    _tc2_body,
    grid=(2, _NB),
    in_specs=[
        _spec_x,
        _spec_x,
        _spec_dis,
        pl.BlockSpec((D, D), lambda c, j: (0, 0)),
        pl.BlockSpec((1, D), lambda c, j: (0, 0)),
    ],
    out_specs=_spec_x,
    out_shape=jax.ShapeDtypeStruct((2, N, D), jnp.float32),
)

_tc3 = pl.pallas_call(
    _tc3_body,
    grid=(2, _NB),
    in_specs=[
        _spec_x,
        _spec_x,
        _spec_dis,
        pl.BlockSpec(memory_space=pltpu.SMEM),
    ],
    out_specs=pl.BlockSpec((_BR, D), lambda c, j: (j, c)),
    out_shape=jax.ShapeDtypeStruct((N, 2 * D), jnp.float32),
)


def kernel(x, edge_index, W1, b1, W2, b2, Wo, bo, weight):
    row = edge_index[0]
    col = edge_index[1]

    col_d = jnp.pad(col.reshape(32, E // 32), ((0, 0), (0, DEG_NCH * CH - E // 32)),
                    constant_values=N).reshape(32 * DEG_NCH, CH)
    r16 = jnp.pad(row.reshape(16, E // 16), ((0, 0), (0, AGG_NCH * ACH - E // 16)))
    c16 = jnp.pad(col.reshape(16, E // 16), ((0, 0), (0, AGG_NCH * ACH - E // 16)),
                  constant_values=N)
    row2 = jnp.concatenate([r16, r16 + N], axis=0).reshape(32 * AGG_NCH, ACH)
    colh = c16.reshape(16 * AGG_NCH, ACH)

    ones128 = jnp.ones((CH, D), jnp.float32)
    zeros128 = jnp.zeros((CH, D), jnp.float32)
    zeros64 = jnp.zeros((ACH, D), jnp.float32)

    degtab = _deg_call(col_d, ones128, zeros128).reshape(2, DEG_ROWS, D)

    Wst = jnp.stack([W1, W2])
    bst = jnp.stack([b1, b2]).reshape(2, 1, D)
    G1, dis = _tc1(x, Wst, bst, degtab, degtab)
    A1 = _agg_call(G1.reshape(2 * N, D), row2, colh, zeros64)
    Agg1 = A1.reshape(2, ACC_ROWS, D)

    G2 = _tc2(Agg1, G1, dis, Wo, bo.reshape(1, D))
    A2 = _agg_call(G2.reshape(2 * N, D), row2, colh, zeros64)
    Agg2 = A2.reshape(2, ACC_ROWS, D)

    return _tc3(Agg2, G2, dis, weight)
